# baseline (speedup 1.0000x reference)
import functools

import numpy as np
import jax
import jax.numpy as jnp
from jax import lax
from jax.experimental import pallas as pl
from jax.experimental.pallas import tpu as pltpu
from jax.experimental.pallas import tpu_sc as plsc

_VOCAB = 1000000
_D = 64
_B = 4096
_NS = 128
_TOTAL = _B * _NS

_NC = 2
_NSUB = 16
_NW = _NC * _NSUB
_ROWS_PER_W = _TOTAL // _NW
_CHUNK = 512
_NCHUNKS = _ROWS_PER_W // _CHUNK
_NRING = 3
_L = 16


def _np_threefry2x32(k1, k2, x1, x2):
    r0, r1 = (13, 15, 26, 6), (17, 29, 16, 24)
    ks = (k1, k2, np.uint32(k1 ^ k2 ^ np.uint32(0x1BD11BDA)))
    a = (x1 + ks[0]).astype(np.uint32)
    b = (x2 + ks[1]).astype(np.uint32)
    for rots, ia, ib, c in ((r0, 1, 2, 1), (r1, 2, 0, 2), (r0, 0, 1, 3),
                            (r1, 1, 2, 4), (r0, 2, 0, 5)):
        for r in rots:
            a = (a + b).astype(np.uint32)
            b = ((b << np.uint32(r)) | (b >> np.uint32(32 - r))).astype(np.uint32)
            b = a ^ b
        a = (a + ks[ia]).astype(np.uint32)
        b = (b + ks[ib] + np.uint32(c)).astype(np.uint32)
    return a, b


def _lower_subkey():
    old = np.seterr(over="ignore")
    try:
        b1, b2 = _np_threefry2x32(np.uint32(0), np.uint32(42),
                                  np.zeros(2, np.uint32),
                                  np.arange(2, dtype=np.uint32))
        return int(b1[1]), int(b2[1])
    finally:
        np.seterr(**old)


_KLO1, _KLO2 = _lower_subkey()
_K3 = _KLO1 ^ _KLO2 ^ 0x1BD11BDA


def _tf_lo16(cnt):
    r0, r1 = (13, 15, 26, 6), (17, 29, 16, 24)
    ks = (jnp.uint32(_KLO1), jnp.uint32(_KLO2), jnp.uint32(_K3))
    a = jnp.full((_L,), _KLO1, jnp.uint32)
    b = cnt + ks[1]
    for rots, ia, ib, c in ((r0, 1, 2, 1), (r1, 2, 0, 2), (r0, 0, 1, 3),
                            (r1, 1, 2, 4), (r0, 2, 0, 5)):
        for r in rots:
            a = a + b
            b = lax.shift_left(b, jnp.uint32(r)) | lax.shift_right_logical(
                b, jnp.uint32(32 - r))
            b = a ^ b
        a = a + ks[ia]
        b = b + ks[ib] + jnp.uint32(c)
    return a ^ b


def _mod_vocab(lo):
    hi = plsc.bitcast(lax.shift_right_logical(lo, jnp.uint32(6)), jnp.int32)
    q = (hi.astype(jnp.float32) * jnp.float32(64.0 / _VOCAB)).astype(jnp.int32)
    r = plsc.bitcast(lo, jnp.int32) - q * jnp.int32(_VOCAB)
    for _ in range(2):
        r = jnp.where(r < 0, r + _VOCAB, r)
        r = jnp.where(r >= _VOCAB, r - _VOCAB, r)
    return r


def _rsqrt16(s):
    i = plsc.bitcast(s, jnp.int32)
    i = jnp.int32(0x5F3759DF) - lax.shift_right_logical(i, 1)
    y = plsc.bitcast(i, jnp.float32)
    for _ in range(2):
        y = y * (jnp.float32(1.5) - jnp.float32(0.5) * s * y * y)
    return y


def _body(ids_tab_hbm, table_hbm, ids_out_hbm, emb_out_hbm,
          obuf, ids_all, rows0, rows1, rows2,
          semi, semr0, semr1, semr2, semw0, semw1, semw2):
    c = lax.axis_index("c")
    s = lax.axis_index("s")
    wid = s * _NC + c
    rbase = wid * _ROWS_PER_W
    lanes = lax.iota(jnp.int32, _L)
    lanes_u = plsc.bitcast(lanes, jnp.uint32)

    def win(w, carry):
        @pl.when(w >= 4)
        def _():
            pltpu.make_async_copy(ids_tab_hbm.at[obuf.at[0]],
                                  ids_all.at[pl.ds(0, _CHUNK)], semi).wait()

        def vec(v, carry2):
            base = rbase + w * _CHUNK + v * _L
            cnt = plsc.bitcast(jnp.full((_L,), base, jnp.int32), jnp.uint32)
            off = _mod_vocab(_tf_lo16(cnt + lanes_u))
            obuf[w % 4, pl.ds(v * _L, _L)] = off
            return carry2

        lax.fori_loop(0, _CHUNK // _L, vec, 0)
        pltpu.async_copy(ids_tab_hbm.at[obuf.at[w % 4]],
                         ids_all.at[pl.ds(w * _CHUNK, _CHUNK)], semi)
        return carry

    lax.fori_loop(0, _NCHUNKS, win, 0)
    for _ in range(4):
        pltpu.make_async_copy(ids_tab_hbm.at[obuf.at[0]],
                              ids_all.at[pl.ds(0, _CHUNK)], semi).wait()
    pltpu.sync_copy(ids_all, ids_out_hbm.at[pl.ds(rbase, _ROWS_PER_W)])

    rows = [rows0, rows1, rows2]
    semr = [semr0, semr1, semr2]
    semw = [semw0, semw1, semw2]

    def fire(cidx, j):
        pltpu.async_copy(table_hbm.at[ids_all.at[pl.ds(cidx * _CHUNK, _CHUNK)]],
                         rows[j], semr[j])

    def drain_rows(j):
        pltpu.make_async_copy(table_hbm.at[ids_all.at[pl.ds(0, _CHUNK)]],
                              rows[j], semr[j]).wait()

    def write_out(cidx, j):
        off = rbase + cidx * _CHUNK
        pltpu.async_copy(rows[j], emb_out_hbm.at[pl.ds(off, _CHUNK)], semw[j])

    def drain_write(j):
        pltpu.make_async_copy(rows[j], emb_out_hbm.at[pl.ds(0, _CHUNK)],
                              semw[j]).wait()

    def norm(j):
        rows_v = rows[j]

        @plsc.parallel_loop(0, _CHUNK, unroll=8)
        def row(r):
            rr = rows_v.at[r]
            vs = [rr[pl.ds(q * _L, _L)] for q in range(_D // _L)]
            acc = vs[0] * vs[0]
            for v in vs[1:]:
                acc = acc + v * v
            sv = jnp.full((_L,), jnp.sum(acc), jnp.float32)
            y = jnp.minimum(_rsqrt16(sv), jnp.float32(1e8))
            for q, v in enumerate(vs):
                rr[pl.ds(q * _L, _L)] = v * y

    fire(0, 0)
    fire(1, 1)

    def ringstep(g, carry):
        for j in range(_NRING):
            cidx = g * _NRING + j
            jn = (j + 2) % _NRING
            if j == 0:
                @pl.when(g > 0)
                def _():
                    drain_write(jn)
            else:
                drain_write(jn)
            fire(cidx + 2, jn)
            drain_rows(j)
            norm(j)
            write_out(cidx, j)
        return carry

    lax.fori_loop(0, (_NCHUNKS - 2) // _NRING, ringstep, 0)
    for cidx, j in ((_NCHUNKS - 2, 0), (_NCHUNKS - 1, 1)):
        drain_write((j + 2) % _NRING)
        drain_rows(j)
        norm(j)
        write_out(cidx, j)
    drain_write(1)


@functools.cache
def _sampler():
    return pl.kernel(
        _body,
        out_type=[
            jax.ShapeDtypeStruct((_TOTAL,), jnp.int32),
            jax.ShapeDtypeStruct((_TOTAL, _D), jnp.float32),
        ],
        mesh=plsc.VectorSubcoreMesh(core_axis_name="c", subcore_axis_name="s"),
        compiler_params=pltpu.CompilerParams(
            needs_layout_passes=False, use_tc_tiling_on_sc=False),
        scratch_types=[
            pltpu.VMEM((4, _CHUNK), jnp.int32),
            pltpu.VMEM((_ROWS_PER_W,), jnp.int32),
            pltpu.VMEM((_CHUNK, _D), jnp.float32),
            pltpu.VMEM((_CHUNK, _D), jnp.float32),
            pltpu.VMEM((_CHUNK, _D), jnp.float32),
        ] + [pltpu.SemaphoreType.DMA] * 7,
    )


def kernel(postive_item_ids, num_to_sample, item_emb_table, all_item_ids):
    del postive_item_ids, num_to_sample
    ids_flat, emb_flat = _sampler()(all_item_ids, item_emb_table)
    return ids_flat.reshape(_B, _NS), emb_flat.reshape(_B, _NS, _D)


# speedup vs baseline: 1.0701x; 1.0701x over previous
import functools

import numpy as np
import jax
import jax.numpy as jnp
from jax import lax
from jax.experimental import pallas as pl
from jax.experimental.pallas import tpu as pltpu
from jax.experimental.pallas import tpu_sc as plsc

_VOCAB = 1000000
_D = 64
_B = 4096
_NS = 128
_TOTAL = _B * _NS

_NC = 2
_NSUB = 16
_NW = _NC * _NSUB
_ROWS_PER_W = _TOTAL // _NW
_CHUNK = 256
_NCHUNKS = _ROWS_PER_W // _CHUNK
_NRING = 4
_L = 16


def _np_threefry2x32(k1, k2, x1, x2):
    r0, r1 = (13, 15, 26, 6), (17, 29, 16, 24)
    ks = (k1, k2, np.uint32(k1 ^ k2 ^ np.uint32(0x1BD11BDA)))
    a = (x1 + ks[0]).astype(np.uint32)
    b = (x2 + ks[1]).astype(np.uint32)
    for rots, ia, ib, c in ((r0, 1, 2, 1), (r1, 2, 0, 2), (r0, 0, 1, 3),
                            (r1, 1, 2, 4), (r0, 2, 0, 5)):
        for r in rots:
            a = (a + b).astype(np.uint32)
            b = ((b << np.uint32(r)) | (b >> np.uint32(32 - r))).astype(np.uint32)
            b = a ^ b
        a = (a + ks[ia]).astype(np.uint32)
        b = (b + ks[ib] + np.uint32(c)).astype(np.uint32)
    return a, b


def _lower_subkey():
    old = np.seterr(over="ignore")
    try:
        b1, b2 = _np_threefry2x32(np.uint32(0), np.uint32(42),
                                  np.zeros(2, np.uint32),
                                  np.arange(2, dtype=np.uint32))
        return int(b1[1]), int(b2[1])
    finally:
        np.seterr(**old)


_KLO1, _KLO2 = _lower_subkey()
_K3 = _KLO1 ^ _KLO2 ^ 0x1BD11BDA


def _tf_lo16(cnt):
    r0, r1 = (13, 15, 26, 6), (17, 29, 16, 24)
    ks = (jnp.uint32(_KLO1), jnp.uint32(_KLO2), jnp.uint32(_K3))
    a = jnp.full((_L,), _KLO1, jnp.uint32)
    b = cnt + ks[1]
    for rots, ia, ib, c in ((r0, 1, 2, 1), (r1, 2, 0, 2), (r0, 0, 1, 3),
                            (r1, 1, 2, 4), (r0, 2, 0, 5)):
        for r in rots:
            a = a + b
            b = lax.shift_left(b, jnp.uint32(r)) | lax.shift_right_logical(
                b, jnp.uint32(32 - r))
            b = a ^ b
        a = a + ks[ia]
        b = b + ks[ib] + jnp.uint32(c)
    return a ^ b


def _mod_vocab(lo):
    hi = plsc.bitcast(lax.shift_right_logical(lo, jnp.uint32(6)), jnp.int32)
    q = (hi.astype(jnp.float32) * jnp.float32(64.0 / _VOCAB)).astype(jnp.int32)
    r = plsc.bitcast(lo, jnp.int32) - q * jnp.int32(_VOCAB)
    for _ in range(2):
        r = jnp.where(r < 0, r + _VOCAB, r)
        r = jnp.where(r >= _VOCAB, r - _VOCAB, r)
    return r


def _rsqrt16(s):
    i = plsc.bitcast(s, jnp.int32)
    i = jnp.int32(0x5F3759DF) - lax.shift_right_logical(i, 1)
    y = plsc.bitcast(i, jnp.float32)
    for _ in range(2):
        y = y * (jnp.float32(1.5) - jnp.float32(0.5) * s * y * y)
    return y


def _body_ids(ids_tab_hbm, ids_out_hbm, obuf, ids_all, semi):
    c = lax.axis_index("c")
    s = lax.axis_index("s")
    wid = s * _NC + c
    rbase = wid * _ROWS_PER_W
    lanes = lax.iota(jnp.int32, _L)
    lanes_u = plsc.bitcast(lanes, jnp.uint32)

    def win(w, carry):
        @pl.when(w >= 4)
        def _():
            pltpu.make_async_copy(ids_tab_hbm.at[obuf.at[0]],
                                  ids_all.at[pl.ds(0, _CHUNK)], semi).wait()

        def vec(v, carry2):
            base = rbase + w * _CHUNK + v * _L
            cnt = plsc.bitcast(jnp.full((_L,), base, jnp.int32), jnp.uint32)
            off = _mod_vocab(_tf_lo16(cnt + lanes_u))
            obuf[w % 4, pl.ds(v * _L, _L)] = off
            return carry2

        lax.fori_loop(0, _CHUNK // _L, vec, 0)
        pltpu.async_copy(ids_tab_hbm.at[obuf.at[w % 4]],
                         ids_all.at[pl.ds(w * _CHUNK, _CHUNK)], semi)
        return carry

    lax.fori_loop(0, _NCHUNKS, win, 0)
    for _ in range(4):
        pltpu.make_async_copy(ids_tab_hbm.at[obuf.at[0]],
                              ids_all.at[pl.ds(0, _CHUNK)], semi).wait()
    pltpu.sync_copy(ids_all, ids_out_hbm.at[pl.ds(rbase, _ROWS_PER_W)])


def _body_rows(ids_in_hbm, table_hbm, emb_out_hbm,
               ids_all, rows0, rows1, rows2, rows3,
               semr0, semr1, semr2, semr3, semw0, semw1, semw2, semw3):
    c = lax.axis_index("c")
    s = lax.axis_index("s")
    wid = s * _NC + c
    rbase = wid * _ROWS_PER_W
    pltpu.sync_copy(ids_in_hbm.at[pl.ds(rbase, _ROWS_PER_W)], ids_all)

    rows = [rows0, rows1, rows2, rows3]
    semr = [semr0, semr1, semr2, semr3]
    semw = [semw0, semw1, semw2, semw3]

    def fire(cidx, j):
        pltpu.async_copy(table_hbm.at[ids_all.at[pl.ds(cidx * _CHUNK, _CHUNK)]],
                         rows[j], semr[j])

    def drain_rows(j):
        pltpu.make_async_copy(table_hbm.at[ids_all.at[pl.ds(0, _CHUNK)]],
                              rows[j], semr[j]).wait()

    def write_out(cidx, j):
        off = rbase + cidx * _CHUNK
        pltpu.async_copy(rows[j], emb_out_hbm.at[pl.ds(off, _CHUNK)], semw[j])

    def drain_write(j):
        pltpu.make_async_copy(rows[j], emb_out_hbm.at[pl.ds(0, _CHUNK)],
                              semw[j]).wait()

    def norm(j):
        rows_v = rows[j]

        @plsc.parallel_loop(0, _CHUNK, unroll=8)
        def row(r):
            rr = rows_v.at[r]
            vs = [rr[pl.ds(q * _L, _L)] for q in range(_D // _L)]
            acc = vs[0] * vs[0]
            for v in vs[1:]:
                acc = acc + v * v
            sv = jnp.full((_L,), jnp.sum(acc), jnp.float32)
            y = jnp.minimum(_rsqrt16(sv), jnp.float32(1e8))
            for q, v in enumerate(vs):
                rr[pl.ds(q * _L, _L)] = v * y

    fire(0, 0)
    fire(1, 1)

    def ringstep(g, carry):
        for j in range(_NRING):
            cidx = g * _NRING + j
            jn = (j + 2) % _NRING
            if j < 2:
                @pl.when(g > 0)
                def _():
                    drain_write(jn)
                fire(cidx + 2, jn)
            else:
                drain_write(jn)

                @pl.when(g < _NCHUNKS // _NRING - 1)
                def _():
                    fire(cidx + 2, jn)
            drain_rows(j)
            norm(j)
            write_out(cidx, j)
        return carry

    lax.fori_loop(0, _NCHUNKS // _NRING, ringstep, 0)
    drain_write(2)
    drain_write(3)


@functools.cache
def _sampler_ids():
    return pl.kernel(
        _body_ids,
        out_type=[jax.ShapeDtypeStruct((_TOTAL,), jnp.int32)],
        mesh=plsc.VectorSubcoreMesh(core_axis_name="c", subcore_axis_name="s"),
        compiler_params=pltpu.CompilerParams(
            needs_layout_passes=False, use_tc_tiling_on_sc=False),
        scratch_types=[
            pltpu.VMEM((4, _CHUNK), jnp.int32),
            pltpu.VMEM((_ROWS_PER_W,), jnp.int32),
            pltpu.SemaphoreType.DMA,
        ],
    )


@functools.cache
def _sampler_rows():
    return pl.kernel(
        _body_rows,
        out_type=[jax.ShapeDtypeStruct((_TOTAL, _D), jnp.float32)],
        mesh=plsc.VectorSubcoreMesh(core_axis_name="c", subcore_axis_name="s"),
        compiler_params=pltpu.CompilerParams(
            needs_layout_passes=False, use_tc_tiling_on_sc=False),
        scratch_types=[
            pltpu.VMEM((_ROWS_PER_W,), jnp.int32),
            pltpu.VMEM((_CHUNK, _D), jnp.float32),
            pltpu.VMEM((_CHUNK, _D), jnp.float32),
            pltpu.VMEM((_CHUNK, _D), jnp.float32),
            pltpu.VMEM((_CHUNK, _D), jnp.float32),
        ] + [pltpu.SemaphoreType.DMA] * 8,
    )


def kernel(postive_item_ids, num_to_sample, item_emb_table, all_item_ids):
    del postive_item_ids, num_to_sample
    (ids_flat,) = _sampler_ids()(all_item_ids)
    (emb_flat,) = _sampler_rows()(ids_flat, item_emb_table)
    return ids_flat.reshape(_B, _NS), emb_flat.reshape(_B, _NS, _D)


# speedup vs baseline: 1.2908x; 1.2062x over previous
import functools

import numpy as np
import jax
import jax.numpy as jnp
from jax import lax
from jax.experimental import pallas as pl
from jax.experimental.pallas import tpu as pltpu
from jax.experimental.pallas import tpu_sc as plsc

_VOCAB = 1000000
_D = 64
_B = 4096
_NS = 128
_TOTAL = _B * _NS

_NC = 2
_NSUB = 16
_NW = _NC * _NSUB
_ROWS_PER_W = _TOTAL // _NW
_CHUNK = 256
_NCHUNKS = _ROWS_PER_W // _CHUNK
_NRING = 4
_IDXW = 128
_OTR = (_CHUNK // _IDXW) * _D
_L = 16


def _np_threefry2x32(k1, k2, x1, x2):
    r0, r1 = (13, 15, 26, 6), (17, 29, 16, 24)
    ks = (k1, k2, np.uint32(k1 ^ k2 ^ np.uint32(0x1BD11BDA)))
    a = (x1 + ks[0]).astype(np.uint32)
    b = (x2 + ks[1]).astype(np.uint32)
    for rots, ia, ib, c in ((r0, 1, 2, 1), (r1, 2, 0, 2), (r0, 0, 1, 3),
                            (r1, 1, 2, 4), (r0, 2, 0, 5)):
        for r in rots:
            a = (a + b).astype(np.uint32)
            b = ((b << np.uint32(r)) | (b >> np.uint32(32 - r))).astype(np.uint32)
            b = a ^ b
        a = (a + ks[ia]).astype(np.uint32)
        b = (b + ks[ib] + np.uint32(c)).astype(np.uint32)
    return a, b


def _lower_subkey():
    old = np.seterr(over="ignore")
    try:
        b1, b2 = _np_threefry2x32(np.uint32(0), np.uint32(42),
                                  np.zeros(2, np.uint32),
                                  np.arange(2, dtype=np.uint32))
        return int(b1[1]), int(b2[1])
    finally:
        np.seterr(**old)


_KLO1, _KLO2 = _lower_subkey()
_K3 = _KLO1 ^ _KLO2 ^ 0x1BD11BDA


def _tf_lo16(cnt):
    r0, r1 = (13, 15, 26, 6), (17, 29, 16, 24)
    ks = (jnp.uint32(_KLO1), jnp.uint32(_KLO2), jnp.uint32(_K3))
    a = jnp.full((_L,), _KLO1, jnp.uint32)
    b = cnt + ks[1]
    for rots, ia, ib, c in ((r0, 1, 2, 1), (r1, 2, 0, 2), (r0, 0, 1, 3),
                            (r1, 1, 2, 4), (r0, 2, 0, 5)):
        for r in rots:
            a = a + b
            b = lax.shift_left(b, jnp.uint32(r)) | lax.shift_right_logical(
                b, jnp.uint32(32 - r))
            b = a ^ b
        a = a + ks[ia]
        b = b + ks[ib] + jnp.uint32(c)
    return a ^ b


def _mod_vocab(lo):
    hi = plsc.bitcast(lax.shift_right_logical(lo, jnp.uint32(6)), jnp.int32)
    q = (hi.astype(jnp.float32) * jnp.float32(64.0 / _VOCAB)).astype(jnp.int32)
    r = plsc.bitcast(lo, jnp.int32) - q * jnp.int32(_VOCAB)
    for _ in range(2):
        r = jnp.where(r < 0, r + _VOCAB, r)
        r = jnp.where(r >= _VOCAB, r - _VOCAB, r)
    return r


def _rsqrt16(s):
    i = plsc.bitcast(s, jnp.int32)
    i = jnp.int32(0x5F3759DF) - lax.shift_right_logical(i, 1)
    y = plsc.bitcast(i, jnp.float32)
    for _ in range(2):
        y = y * (jnp.float32(1.5) - jnp.float32(0.5) * s * y * y)
    return y


def _body_ids(ids_tab_hbm, ids_out_hbm, obuf, ids_all, semi):
    c = lax.axis_index("c")
    s = lax.axis_index("s")
    wid = s * _NC + c
    rbase = wid * _ROWS_PER_W
    lanes = lax.iota(jnp.int32, _L)
    lanes_u = plsc.bitcast(lanes, jnp.uint32)

    def win(w, carry):
        @pl.when(w >= 4)
        def _():
            pltpu.make_async_copy(ids_tab_hbm.at[obuf.at[0]],
                                  ids_all.at[pl.ds(0, _CHUNK)], semi).wait()

        def vec(v, carry2):
            base = rbase + w * _CHUNK + v * _L
            cnt = plsc.bitcast(jnp.full((_L,), base, jnp.int32), jnp.uint32)
            off = _mod_vocab(_tf_lo16(cnt + lanes_u))
            obuf[w % 4, pl.ds(v * _L, _L)] = off
            return carry2

        lax.fori_loop(0, _CHUNK // _L, vec, 0)
        pltpu.async_copy(ids_tab_hbm.at[obuf.at[w % 4]],
                         ids_all.at[pl.ds(w * _CHUNK, _CHUNK)], semi)
        return carry

    lax.fori_loop(0, _NCHUNKS, win, 0)
    for _ in range(4):
        pltpu.make_async_copy(ids_tab_hbm.at[obuf.at[0]],
                              ids_all.at[pl.ds(0, _CHUNK)], semi).wait()
    pltpu.sync_copy(ids_all, ids_out_hbm.at[pl.ds(rbase, _ROWS_PER_W)])


def _body_rows(ids_in_hbm, table_hbm, emb_out_hbm,
               ids_all, rows0, rows1, rows2, rows3, ot0, ot1,
               semr0, semr1, semr2, semr3, semw0, semw1):
    c = lax.axis_index("c")
    s = lax.axis_index("s")
    wid = s * _NC + c
    rbase = wid * _ROWS_PER_W
    lanes = lax.iota(jnp.int32, _L)
    pltpu.sync_copy(ids_in_hbm.at[pl.ds(rbase, _ROWS_PER_W)], ids_all)

    rows = [rows0, rows1, rows2, rows3]
    semr = [semr0, semr1, semr2, semr3]
    ots = [ot0, ot1]
    semw = [semw0, semw1]
    qlanes = [lanes + q * _L for q in range(_D // _L)]

    def fire(cidx, j):
        pltpu.async_copy(table_hbm.at[ids_all.at[pl.ds(cidx * _CHUNK, _CHUNK)]],
                         rows[j], semr[j])

    def drain_rows(j):
        pltpu.make_async_copy(table_hbm.at[ids_all.at[pl.ds(0, _CHUNK)]],
                              rows[j], semr[j]).wait()

    def write_out(cidx, p):
        orow = (rbase + cidx * _CHUNK) // _IDXW * _D
        pltpu.async_copy(ots[p].at[:, pl.ds(0, _IDXW)],
                         emb_out_hbm.at[pl.ds(orow, _OTR)], semw[p])

    def drain_write(p):
        pltpu.make_async_copy(ots[p].at[:, pl.ds(0, _IDXW)],
                              emb_out_hbm.at[pl.ds(0, _OTR)], semw[p]).wait()

    def norm(j, p):
        rows_v = rows[j]
        ot = ots[p]

        @plsc.parallel_loop(0, _CHUNK, unroll=8)
        def row(r):
            rr = rows_v.at[r]
            vs = [rr[pl.ds(q * _L, _L)] for q in range(_D // _L)]
            acc = vs[0] * vs[0]
            for v in vs[1:]:
                acc = acc + v * v
            sv = jnp.full((_L,), jnp.sum(acc), jnp.float32)
            y = jnp.minimum(_rsqrt16(sv), jnp.float32(1e8))
            dbase = lax.shift_right_logical(r, 7) * _D
            colv = jnp.full((_L,), lax.rem(r, _IDXW), jnp.int32)
            for q, v in enumerate(vs):
                plsc.store_scatter(ot, [dbase + qlanes[q], colv], v * y)

    fire(0, 0)
    fire(1, 1)

    def ringstep(g, carry):
        for j in range(_NRING):
            cidx = g * _NRING + j
            p = j % 2
            jn = (j + 2) % _NRING
            if j < 2:
                fire(cidx + 2, jn)

                @pl.when(g > 0)
                def _():
                    drain_write(p)
            else:
                @pl.when(g < _NCHUNKS // _NRING - 1)
                def _():
                    fire(cidx + 2, jn)
                drain_write(p)
            drain_rows(j)
            norm(j, p)
            write_out(cidx, p)
        return carry

    lax.fori_loop(0, _NCHUNKS // _NRING, ringstep, 0)
    drain_write(0)
    drain_write(1)


@functools.cache
def _sampler_ids():
    return pl.kernel(
        _body_ids,
        out_type=[jax.ShapeDtypeStruct((_TOTAL,), jnp.int32)],
        mesh=plsc.VectorSubcoreMesh(core_axis_name="c", subcore_axis_name="s"),
        compiler_params=pltpu.CompilerParams(
            needs_layout_passes=False, use_tc_tiling_on_sc=False),
        scratch_types=[
            pltpu.VMEM((4, _CHUNK), jnp.int32),
            pltpu.VMEM((_ROWS_PER_W,), jnp.int32),
            pltpu.SemaphoreType.DMA,
        ],
    )


@functools.cache
def _sampler_rows():
    return pl.kernel(
        _body_rows,
        out_type=[jax.ShapeDtypeStruct((_TOTAL // _IDXW * _D, _IDXW),
                                       jnp.float32)],
        mesh=plsc.VectorSubcoreMesh(core_axis_name="c", subcore_axis_name="s"),
        compiler_params=pltpu.CompilerParams(
            needs_layout_passes=False, use_tc_tiling_on_sc=False),
        scratch_types=[
            pltpu.VMEM((_ROWS_PER_W,), jnp.int32),
            pltpu.VMEM((_CHUNK, _D), jnp.float32),
            pltpu.VMEM((_CHUNK, _D), jnp.float32),
            pltpu.VMEM((_CHUNK, _D), jnp.float32),
            pltpu.VMEM((_CHUNK, _D), jnp.float32),
            pltpu.VMEM((_OTR, _IDXW + 1), jnp.float32),
            pltpu.VMEM((_OTR, _IDXW + 1), jnp.float32),
        ] + [pltpu.SemaphoreType.DMA] * 6,
    )


def kernel(postive_item_ids, num_to_sample, item_emb_table, all_item_ids):
    del postive_item_ids, num_to_sample
    (ids_flat,) = _sampler_ids()(all_item_ids)
    (emb2d,) = _sampler_rows()(ids_flat, item_emb_table)
    emb_t = emb2d.reshape(_TOTAL // _IDXW, _D, _IDXW)
    return ids_flat.reshape(_B, _NS), jnp.swapaxes(emb_t, 1, 2)


# speedup vs baseline: 1.2912x; 1.0003x over previous
import functools

import numpy as np
import jax
import jax.numpy as jnp
from jax import lax
from jax.experimental import pallas as pl
from jax.experimental.pallas import tpu as pltpu
from jax.experimental.pallas import tpu_sc as plsc

_VOCAB = 1000000
_D = 64
_B = 4096
_NS = 128
_TOTAL = _B * _NS

_NC = 2
_NSUB = 16
_NW = _NC * _NSUB
_ROWS_PER_W = _TOTAL // _NW
_CHUNK = 256
_NCHUNKS = _ROWS_PER_W // _CHUNK
_NRING = 4
_IDXW = 128
_OTR = (_CHUNK // _IDXW) * _D
_L = 16


def _np_threefry2x32(k1, k2, x1, x2):
    r0, r1 = (13, 15, 26, 6), (17, 29, 16, 24)
    ks = (k1, k2, np.uint32(k1 ^ k2 ^ np.uint32(0x1BD11BDA)))
    a = (x1 + ks[0]).astype(np.uint32)
    b = (x2 + ks[1]).astype(np.uint32)
    for rots, ia, ib, c in ((r0, 1, 2, 1), (r1, 2, 0, 2), (r0, 0, 1, 3),
                            (r1, 1, 2, 4), (r0, 2, 0, 5)):
        for r in rots:
            a = (a + b).astype(np.uint32)
            b = ((b << np.uint32(r)) | (b >> np.uint32(32 - r))).astype(np.uint32)
            b = a ^ b
        a = (a + ks[ia]).astype(np.uint32)
        b = (b + ks[ib] + np.uint32(c)).astype(np.uint32)
    return a, b


def _lower_subkey():
    old = np.seterr(over="ignore")
    try:
        b1, b2 = _np_threefry2x32(np.uint32(0), np.uint32(42),
                                  np.zeros(2, np.uint32),
                                  np.arange(2, dtype=np.uint32))
        return int(b1[1]), int(b2[1])
    finally:
        np.seterr(**old)


_KLO1, _KLO2 = _lower_subkey()
_K3 = _KLO1 ^ _KLO2 ^ 0x1BD11BDA


def _tf_lo16(cnt):
    r0, r1 = (13, 15, 26, 6), (17, 29, 16, 24)
    ks = (jnp.uint32(_KLO1), jnp.uint32(_KLO2), jnp.uint32(_K3))
    a = jnp.full((_L,), _KLO1, jnp.uint32)
    b = cnt + ks[1]
    for rots, ia, ib, c in ((r0, 1, 2, 1), (r1, 2, 0, 2), (r0, 0, 1, 3),
                            (r1, 1, 2, 4), (r0, 2, 0, 5)):
        for r in rots:
            a = a + b
            b = lax.shift_left(b, jnp.uint32(r)) | lax.shift_right_logical(
                b, jnp.uint32(32 - r))
            b = a ^ b
        a = a + ks[ia]
        b = b + ks[ib] + jnp.uint32(c)
    return a ^ b


def _mod_vocab(lo):
    hi = plsc.bitcast(lax.shift_right_logical(lo, jnp.uint32(6)), jnp.int32)
    q = (hi.astype(jnp.float32) * jnp.float32(64.0 / _VOCAB)).astype(jnp.int32)
    r = plsc.bitcast(lo, jnp.int32) - q * jnp.int32(_VOCAB)
    for _ in range(2):
        r = jnp.where(r < 0, r + _VOCAB, r)
        r = jnp.where(r >= _VOCAB, r - _VOCAB, r)
    return r


def _rsqrt16(s):
    i = plsc.bitcast(s, jnp.int32)
    i = jnp.int32(0x5F3759DF) - lax.shift_right_logical(i, 1)
    y = plsc.bitcast(i, jnp.float32)
    for _ in range(2):
        y = y * (jnp.float32(1.5) - jnp.float32(0.5) * s * y * y)
    return y


def _body_ids(ids_tab_hbm, ids_out_hbm, obuf, ids_all, semi):
    c = lax.axis_index("c")
    s = lax.axis_index("s")
    wid = s * _NC + c
    rbase = wid * _ROWS_PER_W
    lanes = lax.iota(jnp.int32, _L)
    lanes_u = plsc.bitcast(lanes, jnp.uint32)

    def win(w, carry):
        @pl.when(w >= 4)
        def _():
            pltpu.make_async_copy(ids_tab_hbm.at[obuf.at[0]],
                                  ids_all.at[pl.ds(0, _CHUNK)], semi).wait()

        def vec(v, carry2):
            base = rbase + w * _CHUNK + v * _L
            cnt = plsc.bitcast(jnp.full((_L,), base, jnp.int32), jnp.uint32)
            off = _mod_vocab(_tf_lo16(cnt + lanes_u))
            obuf[w % 4, pl.ds(v * _L, _L)] = off
            return carry2

        lax.fori_loop(0, _CHUNK // _L, vec, 0)
        pltpu.async_copy(ids_tab_hbm.at[obuf.at[w % 4]],
                         ids_all.at[pl.ds(w * _CHUNK, _CHUNK)], semi)
        return carry

    lax.fori_loop(0, _NCHUNKS, win, 0)
    for _ in range(4):
        pltpu.make_async_copy(ids_tab_hbm.at[obuf.at[0]],
                              ids_all.at[pl.ds(0, _CHUNK)], semi).wait()
    pltpu.sync_copy(ids_all, ids_out_hbm.at[pl.ds(rbase, _ROWS_PER_W)])


def _body_rows(ids_in_hbm, table_hbm, emb_out_hbm,
               ids_all, rows0, rows1, rows2, rows3, ot0, ot1,
               semr0, semr1, semr2, semr3, semw0, semw1):
    c = lax.axis_index("c")
    s = lax.axis_index("s")
    wid = s * _NC + c
    rbase = wid * _ROWS_PER_W
    lanes = lax.iota(jnp.int32, _L)
    pltpu.sync_copy(ids_in_hbm.at[pl.ds(rbase, _ROWS_PER_W)], ids_all)

    rows = [rows0, rows1, rows2, rows3]
    semr = [semr0, semr1, semr2, semr3]
    ots = [ot0, ot1]
    semw = [semw0, semw1]
    qlanes = [lanes + q * _L for q in range(_D // _L)]

    def fire(cidx, j):
        pltpu.async_copy(table_hbm.at[ids_all.at[pl.ds(cidx * _CHUNK, _CHUNK)]],
                         rows[j], semr[j])

    def drain_rows(j):
        pltpu.make_async_copy(table_hbm.at[ids_all.at[pl.ds(0, _CHUNK)]],
                              rows[j], semr[j]).wait()

    def write_out(cidx, p):
        orow = (rbase + cidx * _CHUNK) // _IDXW * _D
        pltpu.async_copy(ots[p].at[:, pl.ds(0, _IDXW)],
                         emb_out_hbm.at[pl.ds(orow, _OTR)], semw[p])

    def drain_write(p):
        pltpu.make_async_copy(ots[p].at[:, pl.ds(0, _IDXW)],
                              emb_out_hbm.at[pl.ds(0, _OTR)], semw[p]).wait()

    def norm(j, p):
        rows_v = rows[j]
        ot = ots[p]

        @plsc.parallel_loop(0, _CHUNK, unroll=8)
        def row(r):
            rr = rows_v.at[r]
            vs = [rr[pl.ds(q * _L, _L)] for q in range(_D // _L)]
            acc = vs[0] * vs[0]
            for v in vs[1:]:
                acc = acc + v * v
            sv = jnp.full((_L,), jnp.sum(acc), jnp.float32)
            y = jnp.minimum(_rsqrt16(sv), jnp.float32(1e8))
            dbase = lax.shift_right_logical(r, 7) * _D
            colv = jnp.full((_L,), lax.rem(r, _IDXW), jnp.int32)
            for q, v in enumerate(vs):
                plsc.store_scatter(ot, [dbase + qlanes[q], colv], v * y)

    fire(0, 0)
    fire(1, 1)
    fire(2, 2)

    def ringstep(g, carry):
        for j in range(_NRING):
            cidx = g * _NRING + j
            p = j % 2
            jn = (j + 3) % _NRING
            if j == 0:
                fire(cidx + 3, jn)

                @pl.when(g > 0)
                def _():
                    drain_write(p)
            else:
                @pl.when(g < _NCHUNKS // _NRING - 1)
                def _():
                    fire(cidx + 3, jn)
                if j == 1:
                    @pl.when(g > 0)
                    def _():
                        drain_write(p)
                else:
                    drain_write(p)
            drain_rows(j)
            norm(j, p)
            write_out(cidx, p)
        return carry

    lax.fori_loop(0, _NCHUNKS // _NRING, ringstep, 0)
    drain_write(0)
    drain_write(1)


@functools.cache
def _sampler_ids():
    return pl.kernel(
        _body_ids,
        out_type=[jax.ShapeDtypeStruct((_TOTAL,), jnp.int32)],
        mesh=plsc.VectorSubcoreMesh(core_axis_name="c", subcore_axis_name="s"),
        compiler_params=pltpu.CompilerParams(
            needs_layout_passes=False, use_tc_tiling_on_sc=False),
        scratch_types=[
            pltpu.VMEM((4, _CHUNK), jnp.int32),
            pltpu.VMEM((_ROWS_PER_W,), jnp.int32),
            pltpu.SemaphoreType.DMA,
        ],
    )


@functools.cache
def _sampler_rows():
    return pl.kernel(
        _body_rows,
        out_type=[jax.ShapeDtypeStruct((_TOTAL // _IDXW * _D, _IDXW),
                                       jnp.float32)],
        mesh=plsc.VectorSubcoreMesh(core_axis_name="c", subcore_axis_name="s"),
        compiler_params=pltpu.CompilerParams(
            needs_layout_passes=False, use_tc_tiling_on_sc=False),
        scratch_types=[
            pltpu.VMEM((_ROWS_PER_W,), jnp.int32),
            pltpu.VMEM((_CHUNK, _D), jnp.float32),
            pltpu.VMEM((_CHUNK, _D), jnp.float32),
            pltpu.VMEM((_CHUNK, _D), jnp.float32),
            pltpu.VMEM((_CHUNK, _D), jnp.float32),
            pltpu.VMEM((_OTR, _IDXW + 1), jnp.float32),
            pltpu.VMEM((_OTR, _IDXW + 1), jnp.float32),
        ] + [pltpu.SemaphoreType.DMA] * 6,
    )


def kernel(postive_item_ids, num_to_sample, item_emb_table, all_item_ids):
    del postive_item_ids, num_to_sample
    (ids_flat,) = _sampler_ids()(all_item_ids)
    (emb2d,) = _sampler_rows()(ids_flat, item_emb_table)
    emb_t = emb2d.reshape(_TOTAL // _IDXW, _D, _IDXW)
    return ids_flat.reshape(_B, _NS), jnp.swapaxes(emb_t, 1, 2)
